# Initial kernel scaffold; baseline (speedup 1.0000x reference)
import jax
import jax.numpy as jnp
from jax.experimental import pallas as pl


def kernel(x, from_idx, to_idx, promo_idx, norm_weight, Wf, bf, Wt, bt, promo_bias):
    raise NotImplementedError("write your pallas kernel here")


# speedup vs baseline: 4.0811x; 4.0811x over previous
import functools

import jax
import jax.numpy as jnp
from jax import lax
from jax.experimental import pallas as pl
from jax.experimental.pallas import tpu as pltpu
from jax.experimental.pallas import tpu_sc as plsc

_B, _C, _W, _H, _D, _V = 64, 256, 16, 16, 64, 8192
_HW = _W * _H
_EPS = 1e-6
_NC, _NS, _L = 2, 16, 16
_NW = _NC * _NS
_BPW = _B // _NW
_CHUNKS = _V // _L


def _tc_gram_body(x_ref, nw_ref, wf_ref, bf_ref, wt_ref, bt_ref, g_ref):
    xb = x_ref[0]
    ms = jnp.mean(xb * xb)
    scale = lax.rsqrt(ms + _EPS)
    y = xb * nw_ref[...] * scale
    f = jnp.dot(wf_ref[...], y, preferred_element_type=jnp.float32) + bf_ref[...]
    t = jnp.dot(wt_ref[...], y, preferred_element_type=jnp.float32) + bt_ref[...]
    g_ref[0] = lax.dot_general(f, t, (((0,), (0,)), ((), ())),
                               preferred_element_type=jnp.float32)


def _tc_gram(x3, nw2, wf, bf2, wt, bt2):
    return pl.pallas_call(
        _tc_gram_body,
        grid=(_B,),
        in_specs=[
            pl.BlockSpec((1, _C, _HW), lambda b: (b, 0, 0)),
            pl.BlockSpec((_C, _HW), lambda b: (0, 0)),
            pl.BlockSpec((_D, _C), lambda b: (0, 0)),
            pl.BlockSpec((_D, 1), lambda b: (0, 0)),
            pl.BlockSpec((_D, _C), lambda b: (0, 0)),
            pl.BlockSpec((_D, 1), lambda b: (0, 0)),
        ],
        out_specs=pl.BlockSpec((1, _HW, _HW), lambda b: (b, 0, 0)),
        out_shape=jax.ShapeDtypeStruct((_B, _HW, _HW), jnp.float32),
    )(x3, nw2, wf, bf2, wt, bt2)


@functools.partial(
    pl.kernel,
    out_type=jax.ShapeDtypeStruct((_B, _V), jnp.float32),
    mesh=plsc.VectorSubcoreMesh(core_axis_name="c", subcore_axis_name="s"),
    compiler_params=pltpu.CompilerParams(needs_layout_passes=False),
    scratch_types=[
        pltpu.VMEM((_V,), jnp.int32),
        pltpu.VMEM((_V,), jnp.float32),
        pltpu.VMEM((_HW * _HW,), jnp.float32),
        pltpu.VMEM((_V,), jnp.float32),
        pltpu.VMEM((_V,), jnp.int32),
        pltpu.VMEM((_V,), jnp.int32),
        pltpu.VMEM((_L,), jnp.float32),
    ],
)
def _sc_score(g_hbm, from_hbm, to_hbm, promo_hbm, pb_hbm, out_hbm,
              flat_v, pbm_v, tab_v, out_v, a_v, b_v, pb_v):
    wid = lax.axis_index("s") * _NC + lax.axis_index("c")
    pltpu.sync_copy(from_hbm, a_v)
    pltpu.sync_copy(to_hbm, b_v)
    pltpu.sync_copy(pb_hbm, pb_v)

    def prep(i, carry):
        sl = pl.ds(i * _L, _L)
        flat_v[sl] = a_v[sl] * _HW + b_v[sl]
        return carry

    lax.fori_loop(0, _CHUNKS, prep, 0)

    pltpu.sync_copy(promo_hbm, b_v)

    def prep_bias(i, carry):
        sl = pl.ds(i * _L, _L)
        pbm_v[sl] = plsc.load_gather(pb_v, [b_v[sl]])
        return carry

    lax.fori_loop(0, _CHUNKS, prep_bias, 0)

    def per_batch(j, carry):
        bidx = wid * _BPW + j
        pltpu.sync_copy(g_hbm.at[bidx], tab_v)

        def gather(i, c2):
            sl = pl.ds(i * _L, _L)
            out_v[sl] = plsc.load_gather(tab_v, [flat_v[sl]]) + pbm_v[sl]
            return c2

        lax.fori_loop(0, _CHUNKS, gather, 0)
        pltpu.sync_copy(out_v, out_hbm.at[bidx])
        return carry

    lax.fori_loop(0, _BPW, per_batch, 0)


def kernel(x, from_idx, to_idx, promo_idx, norm_weight, Wf, bf, Wt, bt, promo_bias):
    x3 = x.reshape(_B, _C, _HW)
    nw2 = norm_weight.reshape(_C, _HW)
    bf2 = bf.reshape(_D, 1)
    bt2 = bt.reshape(_D, 1)
    g = _tc_gram(x3, nw2, Wf, bf2, Wt, bt2)
    g_flat = g.reshape(_B, _HW * _HW)
    pb16 = jnp.pad(promo_bias, (0, _L - promo_bias.shape[0]))
    return _sc_score(g_flat, from_idx, to_idx, promo_idx, pb16)


# speedup vs baseline: 4.9842x; 1.2213x over previous
import functools

import jax
import jax.numpy as jnp
from jax import lax
from jax.experimental import pallas as pl
from jax.experimental.pallas import tpu as pltpu
from jax.experimental.pallas import tpu_sc as plsc

_B, _C, _W, _H, _D, _V = 64, 256, 16, 16, 64, 8192
_HW = _W * _H
_EPS = 1e-6
_NC, _NS, _L = 2, 16, 16
_NW = _NC * _NS
_BPW = _B // _NW
_CHUNKS = _V // _L


def _tc_gram_body(x_ref, nw_ref, wf_ref, bf_ref, wt_ref, bt_ref, g_ref):
    xb = x_ref[0]
    ms = jnp.mean(xb * xb)
    scale = lax.rsqrt(ms + _EPS)
    y = xb * nw_ref[...] * scale
    f = jnp.dot(wf_ref[...], y, preferred_element_type=jnp.float32) + bf_ref[...]
    t = jnp.dot(wt_ref[...], y, preferred_element_type=jnp.float32) + bt_ref[...]
    g_ref[0] = lax.dot_general(f, t, (((0,), (0,)), ((), ())),
                               preferred_element_type=jnp.float32)


def _tc_gram(x3, nw2, wf, bf2, wt, bt2):
    return pl.pallas_call(
        _tc_gram_body,
        grid=(_B,),
        in_specs=[
            pl.BlockSpec((1, _C, _HW), lambda b: (b, 0, 0)),
            pl.BlockSpec((_C, _HW), lambda b: (0, 0)),
            pl.BlockSpec((_D, _C), lambda b: (0, 0)),
            pl.BlockSpec((_D, 1), lambda b: (0, 0)),
            pl.BlockSpec((_D, _C), lambda b: (0, 0)),
            pl.BlockSpec((_D, 1), lambda b: (0, 0)),
        ],
        out_specs=pl.BlockSpec((1, _HW, _HW), lambda b: (b, 0, 0)),
        out_shape=jax.ShapeDtypeStruct((_B, _HW, _HW), jnp.float32),
    )(x3, nw2, wf, bf2, wt, bt2)


@functools.partial(
    pl.kernel,
    out_type=jax.ShapeDtypeStruct((_B, _V), jnp.float32),
    mesh=plsc.VectorSubcoreMesh(core_axis_name="c", subcore_axis_name="s"),
    compiler_params=pltpu.CompilerParams(needs_layout_passes=False),
    scratch_types=[
        pltpu.VMEM((_HW, _HW), jnp.float32),
        pltpu.VMEM((_V,), jnp.float32),
        pltpu.VMEM((_V,), jnp.int32),
        pltpu.VMEM((_V,), jnp.int32),
        pltpu.VMEM((_V,), jnp.int32),
        pltpu.VMEM((_L,), jnp.float32),
        pltpu.SemaphoreType.DMA,
    ],
)
def _sc_score(g_hbm, from_hbm, to_hbm, promo_hbm, pb_hbm, out_hbm,
              tab_v, out_v, p_v, q_v, r_v, pb_v, sem):
    wid = lax.axis_index("s") * _NC + lax.axis_index("c")
    b0 = wid * _BPW
    cp = pltpu.async_copy(g_hbm.at[b0], tab_v, sem)
    pltpu.sync_copy(from_hbm, p_v)
    pltpu.sync_copy(to_hbm, q_v)
    pltpu.sync_copy(promo_hbm, r_v)
    pltpu.sync_copy(pb_hbm, pb_v)
    cp.wait()

    def gather_batch():
        @plsc.parallel_loop(0, _CHUNKS, unroll=8)
        def _(i):
            sl = pl.ds(i * _L, _L)
            out_v[sl] = (plsc.load_gather(tab_v, [p_v[sl], q_v[sl]])
                         + plsc.load_gather(pb_v, [r_v[sl]]))

    gather_batch()
    pltpu.sync_copy(out_v, out_hbm.at[b0])
    pltpu.sync_copy(g_hbm.at[b0 + 1], tab_v)
    gather_batch()
    pltpu.sync_copy(out_v, out_hbm.at[b0 + 1])


def kernel(x, from_idx, to_idx, promo_idx, norm_weight, Wf, bf, Wt, bt, promo_bias):
    x3 = x.reshape(_B, _C, _HW)
    nw2 = norm_weight.reshape(_C, _HW)
    bf2 = bf.reshape(_D, 1)
    bt2 = bt.reshape(_D, 1)
    g = _tc_gram(x3, nw2, Wf, bf2, Wt, bt2)
    pb16 = jnp.pad(promo_bias, (0, _L - promo_bias.shape[0]))
    return _sc_score(g, from_idx, to_idx, promo_idx, pb16)


# speedup vs baseline: 5.2751x; 1.0584x over previous
import functools

import jax
import jax.numpy as jnp
from jax import lax
from jax.experimental import pallas as pl
from jax.experimental.pallas import tpu as pltpu
from jax.experimental.pallas import tpu_sc as plsc

_B, _C, _W, _H, _D, _V = 64, 256, 16, 16, 64, 8192
_HW = _W * _H
_EPS = 1e-6
_NC, _NS, _L = 2, 16, 16
_NW = _NC * _NS
_BPW = _B // _NW
_CHUNKS = _V // _L


def _tc_gram_body(x_ref, nw_ref, w2_ref, bfr_ref, btc_ref, g_ref):
    xb = x_ref[0]
    y = xb * nw_ref[...]
    ab = jnp.dot(w2_ref[...], y, preferred_element_type=jnp.float32)
    at = ab[:_D].T
    ms = jnp.mean(xb * xb)
    scale = lax.rsqrt(ms + _EPS)
    ft = at * scale + bfr_ref[...]
    t = ab[_D:] * scale + btc_ref[...]
    g_ref[0] = jnp.dot(ft, t, preferred_element_type=jnp.float32)


def _tc_gram(x3, nw2, w2, bfr, btc):
    return pl.pallas_call(
        _tc_gram_body,
        grid=(_B,),
        in_specs=[
            pl.BlockSpec((1, _C, _HW), lambda b: (b, 0, 0)),
            pl.BlockSpec((_C, _HW), lambda b: (0, 0)),
            pl.BlockSpec((2 * _D, _C), lambda b: (0, 0)),
            pl.BlockSpec((1, _D), lambda b: (0, 0)),
            pl.BlockSpec((_D, 1), lambda b: (0, 0)),
        ],
        out_specs=pl.BlockSpec((1, _HW, _HW), lambda b: (b, 0, 0)),
        out_shape=jax.ShapeDtypeStruct((_B, _HW, _HW), jnp.float32),
    )(x3, nw2, w2, bfr, btc)


@functools.partial(
    pl.kernel,
    out_type=jax.ShapeDtypeStruct((_B, _V), jnp.float32),
    mesh=plsc.VectorSubcoreMesh(core_axis_name="c", subcore_axis_name="s"),
    compiler_params=pltpu.CompilerParams(needs_layout_passes=False),
    scratch_types=[
        pltpu.VMEM((_HW, _HW), jnp.float32),
        pltpu.VMEM((_V,), jnp.float32),
        pltpu.VMEM((_V,), jnp.int32),
        pltpu.VMEM((_V,), jnp.int32),
        pltpu.VMEM((_V,), jnp.int32),
        pltpu.VMEM((_L,), jnp.float32),
        pltpu.SemaphoreType.DMA,
    ],
)
def _sc_score(g_hbm, from_hbm, to_hbm, promo_hbm, pb_hbm, out_hbm,
              tab_v, out_v, p_v, q_v, r_v, pb_v, sem):
    wid = lax.axis_index("s") * _NC + lax.axis_index("c")
    b0 = wid * _BPW
    cp = pltpu.async_copy(g_hbm.at[b0], tab_v, sem)
    pltpu.sync_copy(from_hbm, p_v)
    pltpu.sync_copy(to_hbm, q_v)
    pltpu.sync_copy(promo_hbm, r_v)
    pltpu.sync_copy(pb_hbm, pb_v)
    cp.wait()

    def gather_batch():
        @plsc.parallel_loop(0, _CHUNKS, unroll=8)
        def _(i):
            sl = pl.ds(i * _L, _L)
            out_v[sl] = (plsc.load_gather(tab_v, [p_v[sl], q_v[sl]])
                         + plsc.load_gather(pb_v, [r_v[sl]]))

    gather_batch()
    pltpu.sync_copy(out_v, out_hbm.at[b0])
    pltpu.sync_copy(g_hbm.at[b0 + 1], tab_v)
    gather_batch()
    pltpu.sync_copy(out_v, out_hbm.at[b0 + 1])


def kernel(x, from_idx, to_idx, promo_idx, norm_weight, Wf, bf, Wt, bt, promo_bias):
    x3 = x.reshape(_B, _C, _HW)
    nw2 = norm_weight.reshape(_C, _HW)
    w2 = jnp.concatenate([Wf, Wt], axis=0)
    g = _tc_gram(x3, nw2, w2, bf.reshape(1, _D), bt.reshape(_D, 1))
    pb16 = jnp.pad(promo_bias, (0, _L - promo_bias.shape[0]))
    return _sc_score(g, from_idx, to_idx, promo_idx, pb16)


# speedup vs baseline: 7.3689x; 1.3969x over previous
import functools

import jax
import jax.numpy as jnp
from jax import lax
from jax.experimental import pallas as pl
from jax.experimental.pallas import tpu as pltpu
from jax.experimental.pallas import tpu_sc as plsc

_B, _C, _W, _H, _D, _V = 64, 256, 16, 16, 64, 8192
_HW = _W * _H
_EPS = 1e-6
_NC, _NS, _L = 2, 16, 16
_NW = _NC * _NS
_CHUNKS = _V // _L

_NSPLIT = 2
_BPS = _B // _NSPLIT
_TCB = 8


def _tc_gram_body(x_ref, nw_ref, w2t_ref, bfr_ref, btr_ref, g_ref):
    for j in range(_TCB):
        xs = x_ref[j]
        y = xs * nw_ref[...]
        ab = jnp.dot(y, w2t_ref[...], preferred_element_type=jnp.float32)
        ms = jnp.mean(xs * xs)
        scale = lax.rsqrt(ms + _EPS)
        ft = ab[:, :_D] * scale + bfr_ref[...]
        tt = ab[:, _D:] * scale + btr_ref[...]
        g_ref[j] = lax.dot_general(ft, tt, (((1,), (1,)), ((), ())),
                                   preferred_element_type=jnp.float32)


def _tc_gram(xcl, nwcl, w2t, bfr, btr):
    return pl.pallas_call(
        _tc_gram_body,
        grid=(_BPS // _TCB,),
        in_specs=[
            pl.BlockSpec((_TCB, _HW, _C), lambda b: (b, 0, 0)),
            pl.BlockSpec((_HW, _C), lambda b: (0, 0)),
            pl.BlockSpec((_C, 2 * _D), lambda b: (0, 0)),
            pl.BlockSpec((1, _D), lambda b: (0, 0)),
            pl.BlockSpec((1, _D), lambda b: (0, 0)),
        ],
        out_specs=pl.BlockSpec((_TCB, _HW, _HW), lambda b: (b, 0, 0)),
        out_shape=jax.ShapeDtypeStruct((_BPS, _HW, _HW), jnp.float32),
    )(xcl, nwcl, w2t, bfr, btr)


@functools.partial(
    pl.kernel,
    out_type=jax.ShapeDtypeStruct((_BPS, _V), jnp.float32),
    mesh=plsc.VectorSubcoreMesh(core_axis_name="c", subcore_axis_name="s"),
    compiler_params=pltpu.CompilerParams(needs_layout_passes=False),
    scratch_types=[
        pltpu.VMEM((_HW, _HW), jnp.float32),
        pltpu.VMEM((_V,), jnp.float32),
        pltpu.VMEM((_V,), jnp.int32),
        pltpu.VMEM((_V,), jnp.int32),
        pltpu.VMEM((_V,), jnp.int32),
        pltpu.VMEM((_L,), jnp.float32),
        pltpu.SemaphoreType.DMA,
    ],
)
def _sc_score(g_hbm, from_hbm, to_hbm, promo_hbm, pb_hbm, out_hbm,
              tab_v, out_v, p_v, q_v, r_v, pb_v, sem):
    wid = lax.axis_index("s") * _NC + lax.axis_index("c")
    cp = pltpu.async_copy(g_hbm.at[wid], tab_v, sem)
    pltpu.sync_copy(from_hbm, p_v)
    pltpu.sync_copy(to_hbm, q_v)
    pltpu.sync_copy(promo_hbm, r_v)
    pltpu.sync_copy(pb_hbm, pb_v)
    cp.wait()

    @plsc.parallel_loop(0, _CHUNKS, unroll=8)
    def _(i):
        sl = pl.ds(i * _L, _L)
        out_v[sl] = (plsc.load_gather(tab_v, [p_v[sl], q_v[sl]])
                     + plsc.load_gather(pb_v, [r_v[sl]]))

    pltpu.sync_copy(out_v, out_hbm.at[wid])


def kernel(x, from_idx, to_idx, promo_idx, norm_weight, Wf, bf, Wt, bt, promo_bias):
    xcl = jnp.transpose(x.reshape(_B, _C, _HW), (0, 2, 1))
    nwcl = jnp.transpose(norm_weight.reshape(_C, _HW), (1, 0))
    w2t = jnp.concatenate([Wf, Wt], axis=0).T
    bfr = bf.reshape(1, _D)
    btr = bt.reshape(1, _D)
    pb16 = jnp.pad(promo_bias, (0, _L - promo_bias.shape[0]))
    outs = []
    for s in range(_NSPLIT):
        gs = _tc_gram(xcl[s * _BPS:(s + 1) * _BPS], nwcl, w2t, bfr, btr)
        outs.append(_sc_score(gs, from_idx, to_idx, promo_idx, pb16))
    return jnp.concatenate(outs, axis=0)
